# Initial kernel scaffold; baseline (speedup 1.0000x reference)
import jax
import jax.numpy as jnp
from jax.experimental import pallas as pl


def kernel(x, edge_index, edge_attr, aW1, ab1, aW2, ab2, mu_W, mu_b, sig_W, sig_b, con_W, con_b, cW1, cb1, cW2, cb2, v_W, v_b):
    raise NotImplementedError("write your pallas kernel here")


# speedup vs baseline: 4.1502x; 4.1502x over previous
import functools

import jax
import jax.numpy as jnp
from jax import lax
from jax.experimental import pallas as pl
from jax.experimental.pallas import tpu as pltpu
from jax.experimental.pallas import tpu_sc as plsc

N = 100000
E = 1600000
NODE = 10
PAD = 16
NC = 2
NS = 16
NW = NC * NS

ROWS = 2 * E
RPW = ROWS // NW
GK = 5000
GCH = RPW // GK
PW = 8
SROW = 128
SB = 23
EP = NW * SROW * 391
RPT = EP // (NW * SROW)
SCH = RPT // SB

_mesh = functools.partial(
    plsc.VectorSubcoreMesh,
    core_axis_name="c", subcore_axis_name="s",
    num_cores=NC, num_subcores=NS,
)

_sc_params = pltpu.CompilerParams(use_tc_tiling_on_sc=False)


@functools.partial(
    pl.kernel,
    out_type=jax.ShapeDtypeStruct((ROWS, PAD), jnp.float32),
    mesh=_mesh(),
    scratch_types=[
        pltpu.VMEM((GK,), jnp.int32),
        pltpu.VMEM((GK, PAD), jnp.float32),
        pltpu.SemaphoreType.DMA,
    ],
    compiler_params=_sc_params,
)
def _gather_rows(table, idx, out, idx_v, rows_v, sem):
    c = lax.axis_index("c")
    s = lax.axis_index("s")
    base = (s * NC + c) * RPW

    def body(k, carry):
        off = base + k * GK
        pltpu.sync_copy(idx.at[pl.ds(off, GK)], idx_v)
        pltpu.async_copy(table.at[idx_v], rows_v, sem).wait()
        pltpu.sync_copy(rows_v, out.at[pl.ds(off, GK)])
        return carry

    lax.fori_loop(0, GCH, body, 0)


@functools.partial(
    pl.kernel,
    out_type=jax.ShapeDtypeStruct((NC, N, PW), jnp.float32),
    mesh=_mesh(),
    scratch_types=[
        pltpu.VMEM((SB, SROW), jnp.int32),
        pltpu.VMEM((SB * SROW, PW), jnp.float32),
        pltpu.VMEM_SHARED((N, PW), jnp.float32),
        pltpu.SemaphoreType.DMA,
    ],
    compiler_params=_sc_params,
)
def _scatter_add(dst2d, t, zeros, out, idx_v, pay_v, acc, sem):
    c = lax.axis_index("c")
    s = lax.axis_index("s")

    @pl.when(s == 0)
    def _():
        pltpu.sync_copy(zeros, acc)

    plsc.subcore_barrier()
    wid = s * NC + c
    row_base = wid * RPT

    def body(k, carry):
        row_off = row_base + k * SB
        pltpu.sync_copy(dst2d.at[pl.ds(row_off, SB)], idx_v)
        pltpu.sync_copy(t.at[pl.ds(row_off * SROW, SB * SROW)], pay_v)
        handles = []
        for j in range(SB):
            handles.append(pltpu.async_copy(
                pay_v.at[pl.ds(j * SROW, SROW)],
                acc.at[idx_v.at[j]], sem, add=True))
        for h in handles:
            h.wait()
        return carry

    lax.fori_loop(0, SCH, body, 0)
    plsc.subcore_barrier()

    @pl.when(s == 0)
    def _():
        pltpu.sync_copy(acc, out.at[c])


BE = 4000


def _edge_mlp_body(g_ref, attr_ref, w1_ref, b1_ref, wo_ref, c4_ref, t_ref):
    tmp = jnp.concatenate([g_ref[0], g_ref[1], attr_ref[...]], axis=1)
    h = jnp.dot(tmp, w1_ref[...], preferred_element_type=jnp.float32)
    h = jnp.maximum(h + b1_ref[...], 0.0)
    t = jnp.dot(h, wo_ref[...], preferred_element_type=jnp.float32)
    t_ref[...] = t + c4_ref[...]


def _edge_mlp(g3, attr, w1g, b1g, wout, c4):
    return pl.pallas_call(
        _edge_mlp_body,
        grid=(E // BE,),
        in_specs=[
            pl.BlockSpec((2, BE, PAD), lambda i: (0, i, 0)),
            pl.BlockSpec((BE, 2), lambda i: (i, 0)),
            pl.BlockSpec((2 * PAD + 2, 128), lambda i: (0, 0)),
            pl.BlockSpec((1, 128), lambda i: (0, 0)),
            pl.BlockSpec((128, PW), lambda i: (0, 0)),
            pl.BlockSpec((1, PW), lambda i: (0, 0)),
        ],
        out_specs=pl.BlockSpec((BE, PW), lambda i: (i, 0)),
        out_shape=jax.ShapeDtypeStruct((E, PW), jnp.float32),
    )(g3, attr, w1g, b1g, wout, c4)


BN = 2000


def _heads_body(x_ref, acc_ref, wh_ref, bh_ref, vwx_ref, z_ref, v_ref):
    i = pl.program_id(0)
    seg = acc_ref[0] + acc_ref[1]
    z = jnp.dot(x_ref[...], wh_ref[...], preferred_element_type=jnp.float32)
    z = z + seg[:, 0:3] + bh_ref[...]
    z_ref[...] = jnp.maximum(z, 0.0) + jnp.log(1.0 + jnp.exp(-jnp.abs(z)))
    part = jnp.sum(jnp.dot(x_ref[...], vwx_ref[...],
                           preferred_element_type=jnp.float32))
    part = part + jnp.sum(seg[:, 3])

    @pl.when(i == 0)
    def _():
        v_ref[...] = jnp.zeros((1, 1), jnp.float32)

    v_ref[...] += jnp.full((1, 1), part, jnp.float32)


def _heads(x, acc, whead, bhead, vwx):
    return pl.pallas_call(
        _heads_body,
        grid=(N // BN,),
        in_specs=[
            pl.BlockSpec((BN, NODE), lambda i: (i, 0)),
            pl.BlockSpec((2, BN, PW), lambda i: (0, i, 0)),
            pl.BlockSpec((NODE, 3), lambda i: (0, 0)),
            pl.BlockSpec((1, 3), lambda i: (0, 0)),
            pl.BlockSpec((NODE, 1), lambda i: (0, 0)),
        ],
        out_specs=[
            pl.BlockSpec((BN, 3), lambda i: (i, 0)),
            pl.BlockSpec((1, 1), lambda i: (0, 0)),
        ],
        out_shape=[
            jax.ShapeDtypeStruct((N, 3), jnp.float32),
            jax.ShapeDtypeStruct((1, 1), jnp.float32),
        ],
    )(x, acc, whead, bhead, vwx)


def kernel(x, edge_index, edge_attr,
           aW1, ab1, aW2, ab2,
           mu_W, mu_b, sig_W, sig_b, con_W, con_b,
           cW1, cb1, cW2, cb2, v_W, v_b):
    f32 = jnp.float32
    threshold = 1e-12

    con_W_x, con_W_h = con_W[:NODE], con_W[NODE:]
    mu_W_x, mu_W_h = mu_W[:NODE], mu_W[NODE:]
    sig_W_x, sig_W_h = sig_W[:NODE], sig_W[NODE:]
    v_W_x, v_W_h = v_W[:NODE], v_W[NODE:]

    wh3 = jnp.concatenate([con_W_h, mu_W_h, sig_W_h], axis=1)
    wout = jnp.zeros((128, PW), f32)
    wout = wout.at[:64, 0:3].set(aW2 @ wh3)
    wout = wout.at[64:, 3:4].set(cW2 @ v_W_h)
    c4 = jnp.concatenate([ab2 @ wh3, jnp.zeros((PW - 3,), f32)]).reshape(1, PW)

    w1 = jnp.concatenate([aW1, cW1], axis=1)
    b1g = jnp.concatenate([ab1, cb1]).reshape(1, 128)
    w1g = jnp.zeros((2 * PAD + 2, 128), f32)
    w1g = w1g.at[0:NODE].set(w1[0:NODE])
    w1g = w1g.at[PAD:PAD + NODE].set(w1[NODE:2 * NODE])
    w1g = w1g.at[2 * PAD:2 * PAD + 2].set(w1[2 * NODE:])

    whead = jnp.concatenate([con_W_x, mu_W_x, sig_W_x], axis=1)
    bhead = jnp.stack([con_b[0], mu_b[0], sig_b[0]]).reshape(1, 3)

    x_pad = jnp.zeros((N, PAD), f32).at[:, :NODE].set(x)
    idx_flat = edge_index.reshape(ROWS)
    g = _gather_rows(x_pad, idx_flat)
    g3 = g.reshape(2, E, PAD)

    t = _edge_mlp(g3, edge_attr, w1g, b1g, wout, c4)

    zeros = jnp.zeros((N, PW), f32)
    pad = EP - E
    dst2d = jnp.concatenate(
        [edge_index[0], jnp.zeros((pad,), edge_index.dtype)]
    ).reshape(EP // SROW, SROW)
    t_pad = jnp.concatenate([t, jnp.zeros((pad, PW), f32)])
    acc = _scatter_add(dst2d, t_pad, zeros)

    z, vpart = _heads(x, acc, whead, bhead, vwx=v_W_x)

    mu = z[0:1, 1:2] + threshold
    sigma = z[0:1, 2:3] + threshold
    alpha = z[1:, 0:1]
    v = vpart.reshape(1) + E * (cb2 @ v_W_h) + v_b
    return (mu, sigma, alpha, v)
